# Initial kernel scaffold; baseline (speedup 1.0000x reference)
import jax
import jax.numpy as jnp
from jax.experimental import pallas as pl


def kernel(img1, img2):
    raise NotImplementedError("write your pallas kernel here")


# speedup vs baseline: 1.1510x; 1.1510x over previous
import numpy as np
import jax
import jax.numpy as jnp
from jax import lax
from jax.experimental import pallas as pl
from jax.experimental.pallas import tpu as pltpu

_WS = 11
_HALF = _WS // 2
_SIGMA = 1.5
_C1 = 0.01 ** 2
_C2 = 0.03 ** 2


def _gauss_taps():
    x = np.arange(_WS, dtype=np.float64) - _WS // 2
    g = np.exp(-(x * x) / (2.0 * _SIGMA * _SIGMA))
    return (g / g.sum()).astype(np.float32)


def _lane_blur_mat(g, n, n_pad):
    ws = g.shape[0]
    half = ws // 2
    i = np.arange(n)[:, None]
    o = np.arange(n)[None, :]
    j = i - o + half
    band = np.where((j >= 0) & (j < ws), g[np.clip(j, 0, ws - 1)], 0.0)
    m = np.zeros((n, n_pad), np.float32)
    m[:, :n] = band.astype(np.float32)
    return m


def _sublane_blur_mat(g, n):
    return np.ascontiguousarray(_lane_blur_mat(g, n, n).T)


def _make_body(taps, D, H, W, W2, R):
    w_consts = [float(v) for v in taps]

    def body(x_ref, y_ref, tw_ref, ah_ref, map_ref, sum_ref, ring_ref):

        def blur(s):
            xs = x_ref[s]
            ys = y_ref[s]
            q = jnp.concatenate([xs, ys, xs * xs, ys * ys, xs * ys],
                                axis=0)
            t = jnp.dot(q, tw_ref[...],
                        preferred_element_type=jnp.float32)
            ah = ah_ref[...]
            for qi in range(5):
                ring_ref[s % R, qi * H:(qi + 1) * H] = jnp.dot(
                    ah, t[qi * H:(qi + 1) * H],
                    preferred_element_type=jnp.float32)

        for s in range(_HALF + 1):
            blur(s)

        def d_body(d, carry):
            vsum = carry
            acc = None
            for k in range(_WS):
                j = d + (k - _HALF)
                wk = jnp.where(jnp.logical_and(j >= 0, j < D),
                               jnp.float32(w_consts[k]), jnp.float32(0.0))
                slab = ring_ref[jnp.clip(j, 0, D - 1) % R]
                acc = wk * slab if acc is None else acc + wk * slab

            mu1 = acc[0 * H:1 * H]
            mu2 = acc[1 * H:2 * H]
            ex2 = acc[2 * H:3 * H]
            ey2 = acc[3 * H:4 * H]
            exy = acc[4 * H:5 * H]

            mu1_sq = mu1 * mu1
            mu2_sq = mu2 * mu2
            mu1_mu2 = mu1 * mu2
            num = (2.0 * mu1_mu2 + _C1) * (2.0 * (exy - mu1_mu2) + _C2)
            den = ((mu1_sq + mu2_sq + _C1)
                   * ((ex2 - mu1_sq) + (ey2 - mu2_sq) + _C2))
            smap = num * pl.reciprocal(den, approx=True)

            sm = smap[:, :W]
            map_ref[d] = -sm
            vsum = vsum + sm

            s_next = d + _HALF + 1

            @pl.when(s_next < D)
            def _():
                blur(s_next)

            return vsum

        vsum = lax.fori_loop(0, D, d_body, jnp.zeros((H, W), jnp.float32),
                             unroll=2)
        sum_ref[0, 0] = jnp.sum(vsum)

    return body


def _ssim3d(img1, img2):
    N, C, D, H, W = img1.shape
    B = N * C
    g = _gauss_taps()
    W2 = ((W + 127) // 128) * 128
    R = 1 << _WS.bit_length()
    tw = jnp.asarray(_lane_blur_mat(g, W, W2))
    ah = jnp.asarray(_sublane_blur_mat(g, H))

    x = img1.astype(jnp.float32).reshape(B, D, H, W)
    y = img2.astype(jnp.float32).reshape(B, D, H, W)

    body = _make_body(tuple(float(v) for v in g), D, H, W, W2, R)

    neg_map, psums = pl.pallas_call(
        body,
        grid=(B,),
        in_specs=[
            pl.BlockSpec((None, D, H, W), lambda b: (b, 0, 0, 0)),
            pl.BlockSpec((None, D, H, W), lambda b: (b, 0, 0, 0)),
            pl.BlockSpec((W, W2), lambda b: (0, 0)),
            pl.BlockSpec((H, H), lambda b: (0, 0)),
        ],
        out_specs=(
            pl.BlockSpec((None, D, H, W), lambda b: (b, 0, 0, 0)),
            pl.BlockSpec((None, 1, 1), lambda b: (b, 0, 0),
                         memory_space=pltpu.MemorySpace.SMEM),
        ),
        out_shape=(
            jax.ShapeDtypeStruct((B, D, H, W), jnp.float32),
            jax.ShapeDtypeStruct((B, 1, 1), jnp.float32),
        ),
        scratch_shapes=[
            pltpu.VMEM((R, 5 * H, W2), jnp.float32),
        ],
        compiler_params=pltpu.CompilerParams(
            dimension_semantics=("parallel",),
            vmem_limit_bytes=56 * 1024 * 1024,
        ),
    )(x, y, tw, ah)

    mean = jnp.sum(psums) / float(B * D * H * W)
    return 1.0 - mean, neg_map.reshape(N, C, D, H, W)


def kernel(img1, img2):
    return _ssim3d(img1, img2)


# speedup vs baseline: 3.5811x; 3.1114x over previous
import numpy as np
import jax
import jax.numpy as jnp
from jax.experimental import pallas as pl
from jax.experimental.pallas import tpu as pltpu

_WS = 11
_HALF = _WS // 2
_SIGMA = 1.5
_C1 = 0.01 ** 2
_C2 = 0.03 ** 2


def _gauss_taps():
    x = np.arange(_WS, dtype=np.float64) - _WS // 2
    g = np.exp(-(x * x) / (2.0 * _SIGMA * _SIGMA))
    return (g / g.sum()).astype(np.float32)


def _lane_blur_mat(g, n):
    ws = g.shape[0]
    half = ws // 2
    i = np.arange(n)[:, None]
    o = np.arange(n)[None, :]
    j = i - o + half
    band = np.where((j >= 0) & (j < ws), g[np.clip(j, 0, ws - 1)], 0.0)
    return band.astype(np.float32)


def _make_body(taps, D, H, W):
    w_consts = [float(v) for v in taps]
    W2 = 2 * W

    def _swap(a):
        return jnp.concatenate([a[:, W:], a[:, :W]], axis=1)

    def body(xy_ref, tw2_ref, ah_ref, map_ref, sum_ref, buf_ref):

        def blur(s):
            p = xy_ref[s]
            q = jnp.concatenate([p, p * p, p * _swap(p)], axis=0)
            t = jnp.dot(q, tw2_ref[...],
                        preferred_element_type=jnp.float32)
            ah = ah_ref[...]
            for gi in range(3):
                buf_ref[s, gi * H:(gi + 1) * H] = jnp.dot(
                    ah, t[gi * H:(gi + 1) * H],
                    preferred_element_type=jnp.float32)

        for s in range(_HALF + 1):
            blur(s)

        vsum = jnp.zeros((H, W), jnp.float32)
        for d in range(D):
            acc = None
            for k in range(_WS):
                j = d + (k - _HALF)
                if j < 0 or j >= D:
                    continue
                term = w_consts[k] * buf_ref[j]
                acc = term if acc is None else acc + term

            p0 = acc[0 * H:1 * H]
            p1 = acc[1 * H:2 * H]
            p2 = acc[2 * H:3 * H]

            prod = p0 * _swap(p0)
            sq = p0 * p0
            sqs = sq + _swap(sq)
            dif = p1 - sq
            sig = dif + _swap(dif)
            num = (2.0 * prod + _C1) * (2.0 * (p2 - prod) + _C2)
            den = (sqs + _C1) * (sig + _C2)
            smap = num * pl.reciprocal(den, approx=True)

            sm = smap[:, :W]
            map_ref[d] = -sm
            vsum = vsum + sm

            s_next = d + _HALF + 1
            if s_next < D:
                blur(s_next)

        sum_ref[0, 0] = jnp.sum(vsum)

    return body


def _ssim3d(img1, img2):
    N, C, D, H, W = img1.shape
    B = N * C
    g = _gauss_taps()
    tw = _lane_blur_mat(g, W)
    tw2 = np.zeros((2 * W, 2 * W), np.float32)
    tw2[:W, :W] = tw
    tw2[W:, W:] = tw
    ah = np.ascontiguousarray(_lane_blur_mat(g, H).T)

    x = img1.astype(jnp.float32).reshape(B, D, H, W)
    y = img2.astype(jnp.float32).reshape(B, D, H, W)
    xy = jnp.concatenate([x, y], axis=-1)

    body = _make_body(tuple(float(v) for v in g), D, H, W)

    neg_map, psums = pl.pallas_call(
        body,
        grid=(B,),
        in_specs=[
            pl.BlockSpec((None, D, H, 2 * W), lambda b: (b, 0, 0, 0)),
            pl.BlockSpec((2 * W, 2 * W), lambda b: (0, 0)),
            pl.BlockSpec((H, H), lambda b: (0, 0)),
        ],
        out_specs=(
            pl.BlockSpec((None, D, H, W), lambda b: (b, 0, 0, 0)),
            pl.BlockSpec((None, 1, 1), lambda b: (b, 0, 0),
                         memory_space=pltpu.MemorySpace.SMEM),
        ),
        out_shape=(
            jax.ShapeDtypeStruct((B, D, H, W), jnp.float32),
            jax.ShapeDtypeStruct((B, 1, 1), jnp.float32),
        ),
        scratch_shapes=[
            pltpu.VMEM((D, 3 * H, 2 * W), jnp.float32),
        ],
        compiler_params=pltpu.CompilerParams(
            dimension_semantics=("parallel",),
            vmem_limit_bytes=56 * 1024 * 1024,
        ),
    )(jnp.asarray(xy), jnp.asarray(tw2), jnp.asarray(ah))

    mean = jnp.sum(psums) / float(B * D * H * W)
    return 1.0 - mean, neg_map.reshape(N, C, D, H, W)


def kernel(img1, img2):
    return _ssim3d(img1, img2)


# speedup vs baseline: 3.5832x; 1.0006x over previous
import numpy as np
import jax
import jax.numpy as jnp
from jax.experimental import pallas as pl
from jax.experimental.pallas import tpu as pltpu

_WS = 11
_HALF = _WS // 2
_SIGMA = 1.5
_C1 = 0.01 ** 2
_C2 = 0.03 ** 2


def _gauss_taps():
    x = np.arange(_WS, dtype=np.float64) - _WS // 2
    g = np.exp(-(x * x) / (2.0 * _SIGMA * _SIGMA))
    return (g / g.sum()).astype(np.float32)


def _lane_blur_mat(g, n):
    ws = g.shape[0]
    half = ws // 2
    i = np.arange(n)[:, None]
    o = np.arange(n)[None, :]
    j = i - o + half
    band = np.where((j >= 0) & (j < ws), g[np.clip(j, 0, ws - 1)], 0.0)
    return band.astype(np.float32)


def _make_body(taps, D, H, W):
    w_consts = [float(v) for v in taps]
    W2 = 2 * W

    def _swap(a):
        return jnp.concatenate([a[:, W:], a[:, :W]], axis=1)

    def body(xy_ref, tw2_ref, ah_ref, map_ref, sum_ref, buf_ref):

        def blur(s):
            p = xy_ref[s]
            q = jnp.concatenate([p, p * p, p * _swap(p)], axis=0)
            t = jnp.dot(q, tw2_ref[...],
                        preferred_element_type=jnp.float32)
            ah = ah_ref[...]
            for gi in range(3):
                buf_ref[s, gi * H:(gi + 1) * H] = jnp.dot(
                    ah, t[gi * H:(gi + 1) * H],
                    preferred_element_type=jnp.float32)

        for s in range(_HALF + 2):
            blur(s)

        vsum = jnp.zeros((H, W), jnp.float32)
        for d in range(0, D, 2):
            acc = [None, None]
            for j in range(max(0, d - _HALF), min(D, d + _HALF + 2)):
                v = buf_ref[j]
                for i in range(2):
                    k = j - (d + i) + _HALF
                    if 0 <= k < _WS:
                        t = w_consts[k] * v
                        acc[i] = t if acc[i] is None else acc[i] + t

            for i in range(2):
                p0 = acc[i][0 * H:1 * H]
                p1 = acc[i][1 * H:2 * H]
                p2 = acc[i][2 * H:3 * H]

                prod = p0 * _swap(p0)
                sq = p0 * p0
                sqs = sq + _swap(sq)
                dif = p1 - sq
                sig = dif + _swap(dif)
                num = (2.0 * prod + _C1) * (2.0 * (p2 - prod) + _C2)
                den = (sqs + _C1) * (sig + _C2)
                smap = num * pl.reciprocal(den, approx=True)

                sm = smap[:, :W]
                map_ref[d + i] = -sm
                vsum = vsum + sm

            for s_next in (d + _HALF + 2, d + _HALF + 3):
                if s_next < D:
                    blur(s_next)

        sum_ref[0, 0] = jnp.sum(vsum)

    return body


def _ssim3d(img1, img2):
    N, C, D, H, W = img1.shape
    B = N * C
    g = _gauss_taps()
    tw = _lane_blur_mat(g, W)
    tw2 = np.zeros((2 * W, 2 * W), np.float32)
    tw2[:W, :W] = tw
    tw2[W:, W:] = tw
    ah = np.ascontiguousarray(_lane_blur_mat(g, H).T)

    x = img1.astype(jnp.float32).reshape(B, D, H, W)
    y = img2.astype(jnp.float32).reshape(B, D, H, W)
    xy = jnp.concatenate([x, y], axis=-1)

    body = _make_body(tuple(float(v) for v in g), D, H, W)

    neg_map, psums = pl.pallas_call(
        body,
        grid=(B,),
        in_specs=[
            pl.BlockSpec((None, D, H, 2 * W), lambda b: (b, 0, 0, 0)),
            pl.BlockSpec((2 * W, 2 * W), lambda b: (0, 0)),
            pl.BlockSpec((H, H), lambda b: (0, 0)),
        ],
        out_specs=(
            pl.BlockSpec((None, D, H, W), lambda b: (b, 0, 0, 0)),
            pl.BlockSpec((None, 1, 1), lambda b: (b, 0, 0),
                         memory_space=pltpu.MemorySpace.SMEM),
        ),
        out_shape=(
            jax.ShapeDtypeStruct((B, D, H, W), jnp.float32),
            jax.ShapeDtypeStruct((B, 1, 1), jnp.float32),
        ),
        scratch_shapes=[
            pltpu.VMEM((D, 3 * H, 2 * W), jnp.float32),
        ],
        compiler_params=pltpu.CompilerParams(
            dimension_semantics=("parallel",),
            vmem_limit_bytes=56 * 1024 * 1024,
        ),
    )(jnp.asarray(xy), jnp.asarray(tw2), jnp.asarray(ah))

    mean = jnp.sum(psums) / float(B * D * H * W)
    return 1.0 - mean, neg_map.reshape(N, C, D, H, W)


def kernel(img1, img2):
    return _ssim3d(img1, img2)
